# baseline (speedup 1.0000x reference)
import functools

import jax
import jax.numpy as jnp
from jax import lax
from jax.experimental import pallas as pl
from jax.experimental.pallas import tpu as pltpu
from jax.experimental.pallas import tpu_sc as plsc

B = 1024
LC = 200
LQ = 50
LW = 16
WORD_DIM = 128
CHAR_DIM = 64
CHAR_VOCAB = 1000
D_MODEL = 128

N_TOK = B * (LC + LQ)
N_C = B * LC
NW = 32
PER_W = N_TOK // NW
CHUNK = 160
N_IT = PER_W // CHUNK
NCH_C = N_C // CHUNK
GHALF = CHUNK // 2
PACKED = CHAR_DIM // 2
ROWSTRIDE = PACKED + 1
CIDSTRIDE = LW + 1
PCH = PACKED * CHUNK


def _pack_pairs(tab):
    t16 = lax.bitcast_convert_type(tab.astype(jnp.bfloat16), jnp.uint16)
    return lax.bitcast_convert_type(
        t16[:, 0::2].astype(jnp.uint32)
        | (t16[:, 1::2].astype(jnp.uint32) << 16), jnp.int32)


def _sc_gather_pool(cwid3, qwid3, ccid2, qcid2, word_table, ctab_p):
    info = plsc.get_sparse_core_info()
    nc = info.num_cores

    @functools.partial(
        pl.kernel,
        mesh=plsc.VectorSubcoreMesh(core_axis_name="c", subcore_axis_name="s"),
        compiler_params=pltpu.CompilerParams(needs_layout_passes=False),
        out_type=[
            jax.ShapeDtypeStruct((N_TOK, WORD_DIM), jnp.float32),
            jax.ShapeDtypeStruct((N_TOK * PACKED,), jnp.int32),
        ],
        scratch_types=[
            pltpu.VMEM((CHAR_VOCAB * ROWSTRIDE,), jnp.int32),
            pltpu.VMEM((2, 2, GHALF), jnp.int32),
            pltpu.VMEM((2, CHUNK, CIDSTRIDE), jnp.int32),
            pltpu.VMEM((2, CHUNK, WORD_DIM), jnp.float32),
            pltpu.VMEM((2, PCH), jnp.int32),
            pltpu.SemaphoreType.DMA,
            pltpu.SemaphoreType.DMA,
            pltpu.SemaphoreType.DMA,
            pltpu.SemaphoreType.DMA,
            pltpu.SemaphoreType.DMA,
            pltpu.SemaphoreType.DMA,
            pltpu.SemaphoreType.DMA,
            pltpu.SemaphoreType.DMA,
            pltpu.SemaphoreType.DMA,
            pltpu.SemaphoreType.DMA,
        ],
    )
    def k(cwid_hbm, qwid_hbm, ccid_hbm, qcid_hbm, wtab_hbm, ctab_hbm,
          wg_hbm, pool_hbm,
          ctab_v, wid_v, cid_v, rows_v, pool_v,
          s_wid0, s_wid1, s_cid0, s_cid1, s_rows0, s_rows1,
          s_wout0, s_wout1, s_pout0, s_pout1):
        s_wid = (s_wid0, s_wid1)
        s_cid = (s_cid0, s_cid1)
        s_rows = (s_rows0, s_rows1)
        s_wout = (s_wout0, s_wout1)
        s_pout = (s_pout0, s_pout1)
        wid = lax.axis_index("s") * nc + lax.axis_index("c")
        ci0 = wid * N_IT
        pltpu.sync_copy(ctab_hbm, ctab_v)
        iota16 = lax.iota(jnp.int32, 16)

        def start_in(ci, b):
            @pl.when(ci < NCH_C)
            def _c():
                pltpu.async_copy(cwid_hbm.at[ci], wid_v.at[b], s_wid[b])
                pltpu.async_copy(
                    ccid_hbm.at[pl.ds(ci * CHUNK, CHUNK)],
                    cid_v.at[b], s_cid[b])

            @pl.when(ci >= NCH_C)
            def _q():
                pltpu.async_copy(qwid_hbm.at[ci - NCH_C], wid_v.at[b],
                                 s_wid[b])
                pltpu.async_copy(
                    qcid_hbm.at[pl.ds((ci - NCH_C) * CHUNK, CHUNK)],
                    cid_v.at[b], s_cid[b])

        def wait_in(b):
            pltpu.make_async_copy(cwid_hbm.at[0], wid_v.at[b], s_wid[b]).wait()
            pltpu.make_async_copy(ccid_hbm.at[pl.ds(0, CHUNK)],
                                  cid_v.at[b], s_cid[b]).wait()

        def wait_out(b):
            pltpu.make_async_copy(rows_v.at[b],
                                  wg_hbm.at[pl.ds(0, CHUNK)], s_wout[b]).wait()
            pltpu.make_async_copy(pool_v.at[b],
                                  pool_hbm.at[pl.ds(0, PCH)], s_pout[b]).wait()

        start_in(ci0, 0)

        @pl.loop(0, N_IT, step=2)
        def _outer(g):
            for b in (0, 1):
                it = g + b
                ci = ci0 + it
                base = ci * CHUNK
                wait_in(b)

                @pl.when(it + 1 < N_IT)
                def _pf():
                    start_in(ci + 1, 1 - b)

                @pl.when(it >= 2)
                def _drain():
                    wait_out(b)

                cps = [pltpu.async_copy(
                    wtab_hbm.at[wid_v.at[b, h]],
                    rows_v.at[b, pl.ds(h * GHALF, GHALF)], s_rows[b])
                    for h in (0, 1)]

                def tb_body(tb, c2):
                    t0 = tb * 16
                    rowi = iota16 + t0
                    cids = [plsc.load_gather(
                        cid_v.at[b], [rowi, jnp.full((16,), j, jnp.int32)])
                        * ROWSTRIDE for j in range(LW)]
                    for p in range(PACKED):
                        m = plsc.bitcast(
                            plsc.load_gather(ctab_v, [cids[0] + p]),
                            jnp.bfloat16)
                        for j in range(1, LW):
                            m = jnp.maximum(m, plsc.bitcast(
                                plsc.load_gather(ctab_v, [cids[j] + p]),
                                jnp.bfloat16))
                        pool_v[b, pl.ds(p * CHUNK + t0, 16)] = (
                            plsc.bitcast(m, jnp.int32))
                    return c2

                lax.fori_loop(0, CHUNK // 16, tb_body, 0)
                for cp in cps:
                    cp.wait()
                pltpu.async_copy(rows_v.at[b], wg_hbm.at[pl.ds(base, CHUNK)],
                                 s_wout[b])
                pltpu.async_copy(pool_v.at[b],
                                 pool_hbm.at[pl.ds(ci * PCH, PCH)], s_pout[b])

        for b in (0, 1):
            wait_out(b)

    return k(cwid3, qwid3, ccid2, qcid2, word_table, ctab_p)


def _lo_f32(w):
    return lax.bitcast_convert_type(w << 16, jnp.float32)


def _hi_f32(w):
    return lax.bitcast_convert_type(w & jnp.int32(-65536), jnp.float32)


def _tc_project(wgath, pool3d, Ww, Wce, Wco, b2):
    CPB = 16
    TN = CPB * CHUNK
    NBLK_C = N_C // TN

    def mm(wg_ref, pt_ref, ww_ref, wce_ref, wco_ref, b_ref,
           outc_ref, outq_ref):
        i = pl.program_id(0)
        word = jnp.dot(wg_ref[...], ww_ref[...],
                       preferred_element_type=jnp.float32) + b_ref[...]
        dn = (((0,), (0,)), ((), ()))
        for c in range(CPB):
            pt = pt_ref[c]
            ch = lax.dot_general(_lo_f32(pt), wce_ref[...],
                                 dimension_numbers=dn,
                                 preferred_element_type=jnp.float32)
            ch += lax.dot_general(_hi_f32(pt), wco_ref[...],
                                  dimension_numbers=dn,
                                  preferred_element_type=jnp.float32)
            res = word[c * CHUNK:(c + 1) * CHUNK, :] + ch

            @pl.when(i < NBLK_C)
            def _c():
                outc_ref[pl.ds(c * CHUNK, CHUNK), :] = res

            @pl.when(i >= NBLK_C)
            def _q():
                outq_ref[pl.ds(c * CHUNK, CHUNK), :] = res

    return pl.pallas_call(
        mm,
        grid=(N_TOK // TN,),
        in_specs=[
            pl.BlockSpec((TN, WORD_DIM), lambda i: (i, 0)),
            pl.BlockSpec((CPB, PACKED, CHUNK), lambda i: (i, 0, 0)),
            pl.BlockSpec((WORD_DIM, D_MODEL), lambda i: (0, 0)),
            pl.BlockSpec((PACKED, D_MODEL), lambda i: (0, 0)),
            pl.BlockSpec((PACKED, D_MODEL), lambda i: (0, 0)),
            pl.BlockSpec((1, D_MODEL), lambda i: (0, 0)),
        ],
        out_specs=[
            pl.BlockSpec((TN, D_MODEL),
                         lambda i: (jnp.minimum(i, NBLK_C - 1), 0)),
            pl.BlockSpec((TN, D_MODEL),
                         lambda i: (jnp.maximum(i - NBLK_C, 0), 0)),
        ],
        out_shape=[
            jax.ShapeDtypeStruct((N_C, D_MODEL), jnp.float32),
            jax.ShapeDtypeStruct((N_TOK - N_C, D_MODEL), jnp.float32),
        ],
    )(wgath, pool3d, Ww, Wce, Wco, b2)


def kernel(Cwid, Ccid, Qwid, Qcid, word_table, char_table, W_proj, b_proj):
    cwid3 = Cwid.reshape(NCH_C, 2, GHALF).astype(jnp.int32)
    qwid3 = Qwid.reshape(N_TOK // CHUNK - NCH_C, 2, GHALF).astype(jnp.int32)
    ccid2 = jnp.pad(Ccid.reshape(N_C, LW).astype(jnp.int32),
                    ((0, 0), (0, CIDSTRIDE - LW)))
    qcid2 = jnp.pad(Qcid.reshape(N_TOK - N_C, LW).astype(jnp.int32),
                    ((0, 0), (0, CIDSTRIDE - LW)))
    ctp = jnp.pad(_pack_pairs(char_table),
                  ((0, 0), (0, ROWSTRIDE - PACKED))).reshape(-1)

    wgath, pool_lin = _sc_gather_pool(cwid3, qwid3, ccid2, qcid2,
                                      word_table, ctp)
    pool3d = pool_lin.reshape(N_TOK // CHUNK, PACKED, CHUNK)

    Wp_c = W_proj[WORD_DIM:]
    Cf, Qf = _tc_project(wgath, pool3d, W_proj[:WORD_DIM],
                         Wp_c[0::2], Wp_c[1::2], b_proj.reshape(1, D_MODEL))
    return (Cf.reshape(B, LC, D_MODEL), Qf.reshape(B, LQ, D_MODEL))


# speedup vs baseline: 1.4179x; 1.4179x over previous
import functools

import jax
import jax.numpy as jnp
from jax import lax
from jax.experimental import pallas as pl
from jax.experimental.pallas import tpu as pltpu
from jax.experimental.pallas import tpu_sc as plsc

B = 1024
LC = 200
LQ = 50
LW = 16
WORD_DIM = 128
CHAR_DIM = 64
CHAR_VOCAB = 1000
D_MODEL = 128

N_TOK = B * (LC + LQ)
N_C = B * LC
NW = 32
PER_W = N_TOK // NW
CHUNK = 160
N_IT = PER_W // CHUNK
NCH_C = N_C // CHUNK
GHALF = CHUNK // 2
PACKED = CHAR_DIM // 2
ROWSTRIDE = PACKED + 1
CIDSTRIDE = LW + 1
PCH = PACKED * CHUNK


def _pack_pairs(tab):
    t16 = lax.bitcast_convert_type(tab.astype(jnp.bfloat16), jnp.uint16)
    return lax.bitcast_convert_type(
        t16[:, 0::2].astype(jnp.uint32)
        | (t16[:, 1::2].astype(jnp.uint32) << 16), jnp.int32)


def _sc_gather_pool(cwid3, qwid3, ccid2, qcid2, word_table, ctab_p):
    info = plsc.get_sparse_core_info()
    nc = info.num_cores

    @functools.partial(
        pl.kernel,
        mesh=plsc.VectorSubcoreMesh(core_axis_name="c", subcore_axis_name="s"),
        compiler_params=pltpu.CompilerParams(needs_layout_passes=False),
        out_type=[
            jax.ShapeDtypeStruct((N_TOK, WORD_DIM), jnp.float32),
            jax.ShapeDtypeStruct((N_TOK * PACKED,), jnp.int32),
        ],
        scratch_types=[
            pltpu.VMEM((CHAR_VOCAB * ROWSTRIDE,), jnp.int32),
            pltpu.VMEM((2, 2, GHALF), jnp.int32),
            pltpu.VMEM((2, CHUNK, CIDSTRIDE), jnp.int32),
            pltpu.VMEM((2, CHUNK, WORD_DIM), jnp.float32),
            pltpu.VMEM((2, PCH), jnp.int32),
            pltpu.SemaphoreType.DMA,
            pltpu.SemaphoreType.DMA,
            pltpu.SemaphoreType.DMA,
            pltpu.SemaphoreType.DMA,
            pltpu.SemaphoreType.DMA,
            pltpu.SemaphoreType.DMA,
            pltpu.SemaphoreType.DMA,
            pltpu.SemaphoreType.DMA,
            pltpu.SemaphoreType.DMA,
            pltpu.SemaphoreType.DMA,
        ],
    )
    def k(cwid_hbm, qwid_hbm, ccid_hbm, qcid_hbm, wtab_hbm, ctab_hbm,
          wg_hbm, pool_hbm,
          ctab_v, wid_v, cid_v, rows_v, pool_v,
          s_wid0, s_wid1, s_cid0, s_cid1, s_rows0, s_rows1,
          s_wout0, s_wout1, s_pout0, s_pout1):
        s_wid = (s_wid0, s_wid1)
        s_cid = (s_cid0, s_cid1)
        s_rows = (s_rows0, s_rows1)
        s_wout = (s_wout0, s_wout1)
        s_pout = (s_pout0, s_pout1)
        wid = lax.axis_index("s") * nc + lax.axis_index("c")
        ci0 = wid * N_IT
        pltpu.sync_copy(ctab_hbm, ctab_v)
        iota16 = lax.iota(jnp.int32, 16)

        def start_in(ci, b):
            @pl.when(ci < NCH_C)
            def _c():
                pltpu.async_copy(cwid_hbm.at[ci], wid_v.at[b], s_wid[b])
                pltpu.async_copy(
                    ccid_hbm.at[pl.ds(ci * CHUNK, CHUNK)],
                    cid_v.at[b], s_cid[b])

            @pl.when(ci >= NCH_C)
            def _q():
                pltpu.async_copy(qwid_hbm.at[ci - NCH_C], wid_v.at[b],
                                 s_wid[b])
                pltpu.async_copy(
                    qcid_hbm.at[pl.ds((ci - NCH_C) * CHUNK, CHUNK)],
                    cid_v.at[b], s_cid[b])

        def wait_in(b):
            pltpu.make_async_copy(cwid_hbm.at[0], wid_v.at[b], s_wid[b]).wait()
            pltpu.make_async_copy(ccid_hbm.at[pl.ds(0, CHUNK)],
                                  cid_v.at[b], s_cid[b]).wait()

        def wait_out(b):
            pltpu.make_async_copy(rows_v.at[b],
                                  wg_hbm.at[pl.ds(0, CHUNK)], s_wout[b]).wait()
            pltpu.make_async_copy(pool_v.at[b],
                                  pool_hbm.at[pl.ds(0, PCH)], s_pout[b]).wait()

        start_in(ci0, 0)

        @pl.loop(0, N_IT, step=2)
        def _outer(g):
            for b in (0, 1):
                it = g + b
                ci = ci0 + it
                base = ci * CHUNK
                wait_in(b)

                @pl.when(it + 1 < N_IT)
                def _pf():
                    start_in(ci + 1, 1 - b)

                @pl.when(it >= 2)
                def _drain():
                    wait_out(b)

                cps = [pltpu.async_copy(
                    wtab_hbm.at[wid_v.at[b, h]],
                    rows_v.at[b, pl.ds(h * GHALF, GHALF)], s_rows[b])
                    for h in (0, 1)]

                def tb_body(tb, c2):
                    t0 = tb * 16
                    rowi = iota16 + t0
                    cids = [plsc.load_gather(
                        cid_v.at[b], [rowi, jnp.full((16,), j, jnp.int32)])
                        * ROWSTRIDE for j in range(LW)]
                    for p in range(PACKED):
                        m = plsc.bitcast(
                            plsc.load_gather(ctab_v, [cids[0] + p]),
                            jnp.bfloat16)
                        for j in range(1, LW):
                            m = jnp.maximum(m, plsc.bitcast(
                                plsc.load_gather(ctab_v, [cids[j] + p]),
                                jnp.bfloat16))
                        pool_v[b, pl.ds(p * CHUNK + t0, 16)] = (
                            plsc.bitcast(m, jnp.int32))
                    return c2

                lax.fori_loop(0, CHUNK // 16, tb_body, 0)
                for cp in cps:
                    cp.wait()
                pltpu.async_copy(rows_v.at[b], wg_hbm.at[pl.ds(base, CHUNK)],
                                 s_wout[b])
                pltpu.async_copy(pool_v.at[b],
                                 pool_hbm.at[pl.ds(ci * PCH, PCH)], s_pout[b])

        for b in (0, 1):
            wait_out(b)

    return k(cwid3, qwid3, ccid2, qcid2, word_table, ctab_p)


def _lo_f32(w):
    return lax.bitcast_convert_type(w << 16, jnp.float32)


def _hi_f32(w):
    return lax.bitcast_convert_type(w & jnp.int32(-65536), jnp.float32)


def _tc_project(wgath, pool3d, Ww, Wce, Wco, b2, nblk, off):
    CPB = 16
    TN = CPB * CHUNK

    def mm(wg_ref, pt_ref, ww_ref, wce_ref, wco_ref, b_ref, out_ref):
        word = jnp.dot(wg_ref[...], ww_ref[...],
                       preferred_element_type=jnp.float32) + b_ref[...]
        dn = (((0,), (0,)), ((), ()))
        for c in range(CPB):
            pt = pt_ref[c]
            ch = lax.dot_general(_lo_f32(pt), wce_ref[...],
                                 dimension_numbers=dn,
                                 preferred_element_type=jnp.float32)
            ch += lax.dot_general(_hi_f32(pt), wco_ref[...],
                                  dimension_numbers=dn,
                                  preferred_element_type=jnp.float32)
            out_ref[pl.ds(c * CHUNK, CHUNK), :] = (
                word[c * CHUNK:(c + 1) * CHUNK, :] + ch)

    return pl.pallas_call(
        mm,
        grid=(nblk,),
        in_specs=[
            pl.BlockSpec((TN, WORD_DIM), lambda i: (i + off, 0)),
            pl.BlockSpec((CPB, PACKED, CHUNK), lambda i: (i + off, 0, 0)),
            pl.BlockSpec((WORD_DIM, D_MODEL), lambda i: (0, 0)),
            pl.BlockSpec((PACKED, D_MODEL), lambda i: (0, 0)),
            pl.BlockSpec((PACKED, D_MODEL), lambda i: (0, 0)),
            pl.BlockSpec((1, D_MODEL), lambda i: (0, 0)),
        ],
        out_specs=pl.BlockSpec((TN, D_MODEL), lambda i: (i, 0)),
        out_shape=jax.ShapeDtypeStruct((nblk * TN, D_MODEL), jnp.float32),
    )(wgath, pool3d, Ww, Wce, Wco, b2)


def kernel(Cwid, Ccid, Qwid, Qcid, word_table, char_table, W_proj, b_proj):
    cwid3 = Cwid.reshape(NCH_C, 2, GHALF).astype(jnp.int32)
    qwid3 = Qwid.reshape(N_TOK // CHUNK - NCH_C, 2, GHALF).astype(jnp.int32)
    ccid2 = jnp.pad(Ccid.reshape(N_C, LW).astype(jnp.int32),
                    ((0, 0), (0, CIDSTRIDE - LW)))
    qcid2 = jnp.pad(Qcid.reshape(N_TOK - N_C, LW).astype(jnp.int32),
                    ((0, 0), (0, CIDSTRIDE - LW)))
    ctp = jnp.pad(_pack_pairs(char_table),
                  ((0, 0), (0, ROWSTRIDE - PACKED))).reshape(-1)

    wgath, pool_lin = _sc_gather_pool(cwid3, qwid3, ccid2, qcid2,
                                      word_table, ctp)
    pool3d = pool_lin.reshape(N_TOK // CHUNK, PACKED, CHUNK)

    Wp_c = W_proj[WORD_DIM:]
    TN = 16 * CHUNK
    args = (wgath, pool3d, W_proj[:WORD_DIM], Wp_c[0::2], Wp_c[1::2],
            b_proj.reshape(1, D_MODEL))
    Cf = _tc_project(*args, N_C // TN, 0)
    Qf = _tc_project(*args, (N_TOK - N_C) // TN, N_C // TN)
    return (Cf.reshape(B, LC, D_MODEL), Qf.reshape(B, LQ, D_MODEL))


# speedup vs baseline: 1.7126x; 1.2079x over previous
import functools

import jax
import jax.numpy as jnp
from jax import lax
from jax.experimental import pallas as pl
from jax.experimental.pallas import tpu as pltpu
from jax.experimental.pallas import tpu_sc as plsc

B = 1024
LC = 200
LQ = 50
LW = 16
WORD_DIM = 128
CHAR_DIM = 64
CHAR_VOCAB = 1000
D_MODEL = 128

N_TOK = B * (LC + LQ)
N_C = B * LC
NW = 32
PER_W = N_TOK // NW
CHUNK = 160
N_IT = PER_W // CHUNK
NCH_C = N_C // CHUNK
GHALF = CHUNK // 2
PACKED = CHAR_DIM // 2
ROWSTRIDE = PACKED + 1
CIDSTRIDE = LW + 1
PCH = PACKED * CHUNK


def _pack_pairs(tab):
    t16 = lax.bitcast_convert_type(tab.astype(jnp.bfloat16), jnp.uint16)
    return lax.bitcast_convert_type(
        t16[:, 0::2].astype(jnp.uint32)
        | (t16[:, 1::2].astype(jnp.uint32) << 16), jnp.int32)


def _sc_gather_pool(cwid3, qwid3, ccid2, qcid2, word_table, ctab_p):
    info = plsc.get_sparse_core_info()
    nc = info.num_cores

    @functools.partial(
        pl.kernel,
        mesh=plsc.VectorSubcoreMesh(core_axis_name="c", subcore_axis_name="s"),
        compiler_params=pltpu.CompilerParams(needs_layout_passes=False),
        out_type=[
            jax.ShapeDtypeStruct((N_TOK, WORD_DIM), jnp.float32),
            jax.ShapeDtypeStruct((N_TOK * PACKED,), jnp.int32),
        ],
        scratch_types=[
            pltpu.VMEM((CHAR_VOCAB * ROWSTRIDE,), jnp.int32),
            pltpu.VMEM((2, 2, GHALF), jnp.int32),
            pltpu.VMEM((2, CHUNK, LW), jnp.int32),
            pltpu.VMEM((2, CHUNK, WORD_DIM), jnp.float32),
            pltpu.VMEM((2, PCH), jnp.int32),
            pltpu.SemaphoreType.DMA,
            pltpu.SemaphoreType.DMA,
            pltpu.SemaphoreType.DMA,
            pltpu.SemaphoreType.DMA,
            pltpu.SemaphoreType.DMA,
            pltpu.SemaphoreType.DMA,
            pltpu.SemaphoreType.DMA,
            pltpu.SemaphoreType.DMA,
            pltpu.SemaphoreType.DMA,
            pltpu.SemaphoreType.DMA,
        ],
    )
    def k(cwid_hbm, qwid_hbm, ccid_hbm, qcid_hbm, wtab_hbm, ctab_hbm,
          wg_hbm, pool_hbm,
          ctab_v, wid_v, cid_v, rows_v, pool_v,
          s_wid0, s_wid1, s_cid0, s_cid1, s_rows0, s_rows1,
          s_wout0, s_wout1, s_pout0, s_pout1):
        s_wid = (s_wid0, s_wid1)
        s_cid = (s_cid0, s_cid1)
        s_rows = (s_rows0, s_rows1)
        s_wout = (s_wout0, s_wout1)
        s_pout = (s_pout0, s_pout1)
        wid = lax.axis_index("s") * nc + lax.axis_index("c")
        ci0 = wid * N_IT
        pltpu.sync_copy(ctab_hbm, ctab_v)
        iota16 = lax.iota(jnp.int32, 16)

        def start_in(ci, b):
            @pl.when(ci < NCH_C)
            def _c():
                pltpu.async_copy(cwid_hbm.at[ci], wid_v.at[b], s_wid[b])
                pltpu.async_copy(
                    ccid_hbm.at[pl.ds(ci * CHUNK, CHUNK)],
                    cid_v.at[b], s_cid[b])

            @pl.when(ci >= NCH_C)
            def _q():
                pltpu.async_copy(qwid_hbm.at[ci - NCH_C], wid_v.at[b],
                                 s_wid[b])
                pltpu.async_copy(
                    qcid_hbm.at[pl.ds((ci - NCH_C) * CHUNK, CHUNK)],
                    cid_v.at[b], s_cid[b])

        def wait_in(b):
            pltpu.make_async_copy(cwid_hbm.at[0], wid_v.at[b], s_wid[b]).wait()
            pltpu.make_async_copy(ccid_hbm.at[pl.ds(0, CHUNK)],
                                  cid_v.at[b], s_cid[b]).wait()

        def wait_out(b):
            pltpu.make_async_copy(rows_v.at[b],
                                  wg_hbm.at[pl.ds(0, CHUNK)], s_wout[b]).wait()
            pltpu.make_async_copy(pool_v.at[b],
                                  pool_hbm.at[pl.ds(0, PCH)], s_pout[b]).wait()

        start_in(ci0, 0)

        @pl.loop(0, N_IT, step=2)
        def _outer(g):
            for b in (0, 1):
                it = g + b
                ci = ci0 + it
                base = ci * CHUNK
                wait_in(b)

                @pl.when(it + 1 < N_IT)
                def _pf():
                    start_in(ci + 1, 1 - b)

                @pl.when(it >= 2)
                def _drain():
                    wait_out(b)

                cps = [pltpu.async_copy(
                    wtab_hbm.at[wid_v.at[b, h]],
                    rows_v.at[b, pl.ds(h * GHALF, GHALF)], s_rows[b])
                    for h in (0, 1)]

                def tb_body(tb, c2):
                    t0 = tb * 16
                    rowi = iota16 + t0
                    cids = [plsc.load_gather(
                        cid_v.at[b], [rowi, (iota16 + j) & (LW - 1)])
                        * ROWSTRIDE for j in range(LW)]
                    for p in range(PACKED):
                        m = plsc.bitcast(
                            plsc.load_gather(ctab_v, [cids[0] + p]),
                            jnp.bfloat16)
                        for j in range(1, LW):
                            m = jnp.maximum(m, plsc.bitcast(
                                plsc.load_gather(ctab_v, [cids[j] + p]),
                                jnp.bfloat16))
                        pool_v[b, pl.ds(p * CHUNK + t0, 16)] = (
                            plsc.bitcast(m, jnp.int32))
                    return c2

                lax.fori_loop(0, CHUNK // 16, tb_body, 0)
                for cp in cps:
                    cp.wait()
                pltpu.async_copy(rows_v.at[b], wg_hbm.at[pl.ds(base, CHUNK)],
                                 s_wout[b])
                pltpu.async_copy(pool_v.at[b],
                                 pool_hbm.at[pl.ds(ci * PCH, PCH)], s_pout[b])

        for b in (0, 1):
            wait_out(b)

    return k(cwid3, qwid3, ccid2, qcid2, word_table, ctab_p)


def _lo_f32(w):
    return lax.bitcast_convert_type(w << 16, jnp.float32)


def _hi_f32(w):
    return lax.bitcast_convert_type(w & jnp.int32(-65536), jnp.float32)


def _tc_project(wgath, pool3d, Ww, Wce, Wco, b2, nblk, off):
    CPB = 16
    TN = CPB * CHUNK

    def mm(wg_ref, pt_ref, ww_ref, wce_ref, wco_ref, b_ref, out_ref):
        word = jnp.dot(wg_ref[...], ww_ref[...],
                       preferred_element_type=jnp.float32) + b_ref[...]
        dn = (((0,), (0,)), ((), ()))
        for c in range(CPB):
            pt = pt_ref[c]
            ch = lax.dot_general(_lo_f32(pt), wce_ref[...],
                                 dimension_numbers=dn,
                                 preferred_element_type=jnp.float32)
            ch += lax.dot_general(_hi_f32(pt), wco_ref[...],
                                  dimension_numbers=dn,
                                  preferred_element_type=jnp.float32)
            out_ref[pl.ds(c * CHUNK, CHUNK), :] = (
                word[c * CHUNK:(c + 1) * CHUNK, :] + ch)

    return pl.pallas_call(
        mm,
        grid=(nblk,),
        in_specs=[
            pl.BlockSpec((TN, WORD_DIM), lambda i: (i + off, 0)),
            pl.BlockSpec((CPB, PACKED, CHUNK), lambda i: (i + off, 0, 0)),
            pl.BlockSpec((WORD_DIM, D_MODEL), lambda i: (0, 0)),
            pl.BlockSpec((PACKED, D_MODEL), lambda i: (0, 0)),
            pl.BlockSpec((PACKED, D_MODEL), lambda i: (0, 0)),
            pl.BlockSpec((1, D_MODEL), lambda i: (0, 0)),
        ],
        out_specs=pl.BlockSpec((TN, D_MODEL), lambda i: (i, 0)),
        out_shape=jax.ShapeDtypeStruct((nblk * TN, D_MODEL), jnp.float32),
    )(wgath, pool3d, Ww, Wce, Wco, b2)


def kernel(Cwid, Ccid, Qwid, Qcid, word_table, char_table, W_proj, b_proj):
    cwid3 = Cwid.reshape(NCH_C, 2, GHALF).astype(jnp.int32)
    qwid3 = Qwid.reshape(N_TOK // CHUNK - NCH_C, 2, GHALF).astype(jnp.int32)
    ccid2 = Ccid.reshape(N_C, LW).astype(jnp.int32)
    qcid2 = Qcid.reshape(N_TOK - N_C, LW).astype(jnp.int32)
    ctp = jnp.pad(_pack_pairs(char_table),
                  ((0, 0), (0, ROWSTRIDE - PACKED))).reshape(-1)

    wgath, pool_lin = _sc_gather_pool(cwid3, qwid3, ccid2, qcid2,
                                      word_table, ctp)
    pool3d = pool_lin.reshape(N_TOK // CHUNK, PACKED, CHUNK)

    Wp_c = W_proj[WORD_DIM:]
    TN = 16 * CHUNK
    args = (wgath, pool3d, W_proj[:WORD_DIM], Wp_c[0::2], Wp_c[1::2],
            b_proj.reshape(1, D_MODEL))
    Cf = _tc_project(*args, N_C // TN, 0)
    Qf = _tc_project(*args, (N_TOK - N_C) // TN, N_C // TN)
    return (Cf.reshape(B, LC, D_MODEL), Qf.reshape(B, LQ, D_MODEL))


# speedup vs baseline: 1.8959x; 1.1070x over previous
import functools

import jax
import jax.numpy as jnp
from jax import lax
from jax.experimental import pallas as pl
from jax.experimental.pallas import tpu as pltpu
from jax.experimental.pallas import tpu_sc as plsc

B = 1024
LC = 200
LQ = 50
LW = 16
WORD_DIM = 128
CHAR_DIM = 64
CHAR_VOCAB = 1000
D_MODEL = 128

N_TOK = B * (LC + LQ)
N_C = B * LC
NW = 32
PER_W = N_TOK // NW
CHUNK = 160
N_IT = PER_W // CHUNK
NCH_C = N_C // CHUNK
GHALF = CHUNK // 2
PACKED = CHAR_DIM // 2
ROWSTRIDE = PACKED + 1
CIDSTRIDE = LW + 1
PCH = PACKED * CHUNK


def _pack_pairs(tab):
    t16 = lax.bitcast_convert_type(tab.astype(jnp.bfloat16), jnp.uint16)
    return lax.bitcast_convert_type(
        t16[:, 0::2].astype(jnp.uint32)
        | (t16[:, 1::2].astype(jnp.uint32) << 16), jnp.int32)


def _sc_gather_pool(wid3, cid2, word_table, ctab_p, n_tok):
    info = plsc.get_sparse_core_info()
    nc = info.num_cores
    n_it = n_tok // (NW * CHUNK)

    @functools.partial(
        pl.kernel,
        mesh=plsc.VectorSubcoreMesh(core_axis_name="c", subcore_axis_name="s"),
        compiler_params=pltpu.CompilerParams(needs_layout_passes=False),
        out_type=[
            jax.ShapeDtypeStruct((n_tok, WORD_DIM), jnp.float32),
            jax.ShapeDtypeStruct((n_tok * PACKED,), jnp.int32),
        ],
        scratch_types=[
            pltpu.VMEM((CHAR_VOCAB * ROWSTRIDE,), jnp.int32),
            pltpu.VMEM((2, 2, GHALF), jnp.int32),
            pltpu.VMEM((2, CHUNK, LW), jnp.int32),
            pltpu.VMEM((2, CHUNK, WORD_DIM), jnp.float32),
            pltpu.VMEM((2, PCH), jnp.int32),
            pltpu.SemaphoreType.DMA,
            pltpu.SemaphoreType.DMA,
            pltpu.SemaphoreType.DMA,
            pltpu.SemaphoreType.DMA,
            pltpu.SemaphoreType.DMA,
            pltpu.SemaphoreType.DMA,
            pltpu.SemaphoreType.DMA,
            pltpu.SemaphoreType.DMA,
            pltpu.SemaphoreType.DMA,
            pltpu.SemaphoreType.DMA,
        ],
    )
    def k(wid_hbm, cid_hbm, wtab_hbm, ctab_hbm,
          wg_hbm, pool_hbm,
          ctab_v, wid_v, cid_v, rows_v, pool_v,
          s_wid0, s_wid1, s_cid0, s_cid1, s_rows0, s_rows1,
          s_wout0, s_wout1, s_pout0, s_pout1):
        s_wid = (s_wid0, s_wid1)
        s_cid = (s_cid0, s_cid1)
        s_rows = (s_rows0, s_rows1)
        s_wout = (s_wout0, s_wout1)
        s_pout = (s_pout0, s_pout1)
        wid = lax.axis_index("s") * nc + lax.axis_index("c")
        ci0 = wid * n_it
        pltpu.sync_copy(ctab_hbm, ctab_v)
        iota16 = lax.iota(jnp.int32, 16)

        def start_in(ci, b):
            pltpu.async_copy(wid_hbm.at[ci], wid_v.at[b], s_wid[b])
            pltpu.async_copy(cid_hbm.at[pl.ds(ci * CHUNK, CHUNK)],
                             cid_v.at[b], s_cid[b])

        def wait_in(b):
            pltpu.make_async_copy(wid_hbm.at[0], wid_v.at[b], s_wid[b]).wait()
            pltpu.make_async_copy(cid_hbm.at[pl.ds(0, CHUNK)],
                                  cid_v.at[b], s_cid[b]).wait()

        def wait_out(b):
            pltpu.make_async_copy(rows_v.at[b],
                                  wg_hbm.at[pl.ds(0, CHUNK)], s_wout[b]).wait()
            pltpu.make_async_copy(pool_v.at[b],
                                  pool_hbm.at[pl.ds(0, PCH)], s_pout[b]).wait()

        start_in(ci0, 0)

        @pl.loop(0, n_it, step=2)
        def _outer(g):
            for b in (0, 1):
                it = g + b
                ci = ci0 + it
                base = ci * CHUNK
                wait_in(b)

                @pl.when(it + 1 < n_it)
                def _pf():
                    start_in(ci + 1, 1 - b)

                @pl.when(it >= 2)
                def _drain():
                    wait_out(b)

                cps = [pltpu.async_copy(
                    wtab_hbm.at[wid_v.at[b, h]],
                    rows_v.at[b, pl.ds(h * GHALF, GHALF)], s_rows[b])
                    for h in (0, 1)]

                def tb_body(tb, c2):
                    t0 = tb * 16
                    rowi = iota16 + t0
                    cids = [plsc.load_gather(
                        cid_v.at[b], [rowi, (iota16 + j) & (LW - 1)])
                        * ROWSTRIDE for j in range(LW)]
                    for p in range(PACKED):
                        m = plsc.bitcast(
                            plsc.load_gather(ctab_v, [cids[0] + p]),
                            jnp.bfloat16)
                        for j in range(1, LW):
                            m = jnp.maximum(m, plsc.bitcast(
                                plsc.load_gather(ctab_v, [cids[j] + p]),
                                jnp.bfloat16))
                        pool_v[b, pl.ds(p * CHUNK + t0, 16)] = (
                            plsc.bitcast(m, jnp.int32))
                    return c2

                lax.fori_loop(0, CHUNK // 16, tb_body, 0)
                for cp in cps:
                    cp.wait()
                pltpu.async_copy(rows_v.at[b], wg_hbm.at[pl.ds(base, CHUNK)],
                                 s_wout[b])
                pltpu.async_copy(pool_v.at[b],
                                 pool_hbm.at[pl.ds(ci * PCH, PCH)], s_pout[b])

        for b in (0, 1):
            wait_out(b)

    return k(wid3, cid2, word_table, ctab_p)


def _lo_f32(w):
    return lax.bitcast_convert_type(w << 16, jnp.float32)


def _hi_f32(w):
    return lax.bitcast_convert_type(w & jnp.int32(-65536), jnp.float32)


def _tc_project(wgath, pool3d, Ww, Wce, Wco, b2, nblk, off):
    CPB = 16
    TN = CPB * CHUNK

    def mm(wg_ref, pt_ref, ww_ref, wce_ref, wco_ref, b_ref, out_ref):
        word = jnp.dot(wg_ref[...], ww_ref[...],
                       preferred_element_type=jnp.float32) + b_ref[...]
        dn = (((0,), (0,)), ((), ()))
        for c in range(CPB):
            pt = pt_ref[c]
            ch = lax.dot_general(_lo_f32(pt), wce_ref[...],
                                 dimension_numbers=dn,
                                 preferred_element_type=jnp.float32)
            ch += lax.dot_general(_hi_f32(pt), wco_ref[...],
                                  dimension_numbers=dn,
                                  preferred_element_type=jnp.float32)
            out_ref[pl.ds(c * CHUNK, CHUNK), :] = (
                word[c * CHUNK:(c + 1) * CHUNK, :] + ch)

    return pl.pallas_call(
        mm,
        grid=(nblk,),
        in_specs=[
            pl.BlockSpec((TN, WORD_DIM), lambda i: (i + off, 0)),
            pl.BlockSpec((CPB, PACKED, CHUNK), lambda i: (i + off, 0, 0)),
            pl.BlockSpec((WORD_DIM, D_MODEL), lambda i: (0, 0)),
            pl.BlockSpec((PACKED, D_MODEL), lambda i: (0, 0)),
            pl.BlockSpec((PACKED, D_MODEL), lambda i: (0, 0)),
            pl.BlockSpec((1, D_MODEL), lambda i: (0, 0)),
        ],
        out_specs=pl.BlockSpec((TN, D_MODEL), lambda i: (i, 0)),
        out_shape=jax.ShapeDtypeStruct((nblk * TN, D_MODEL), jnp.float32),
    )(wgath, pool3d, Ww, Wce, Wco, b2)


def kernel(Cwid, Ccid, Qwid, Qcid, word_table, char_table, W_proj, b_proj):
    cwid3 = Cwid.reshape(NCH_C, 2, GHALF).astype(jnp.int32)
    qwid3 = Qwid.reshape(N_TOK // CHUNK - NCH_C, 2, GHALF).astype(jnp.int32)
    ccid2 = Ccid.reshape(N_C, LW).astype(jnp.int32)
    qcid2 = Qcid.reshape(N_TOK - N_C, LW).astype(jnp.int32)
    ctp = jnp.pad(_pack_pairs(char_table),
                  ((0, 0), (0, ROWSTRIDE - PACKED))).reshape(-1)

    wg_c, pool_c = _sc_gather_pool(cwid3, ccid2, word_table, ctp, N_C)
    wg_q, pool_q = _sc_gather_pool(qwid3, qcid2, word_table, ctp,
                                   N_TOK - N_C)

    Wp_c = W_proj[WORD_DIM:]
    TN = 16 * CHUNK
    w_args = (W_proj[:WORD_DIM], Wp_c[0::2], Wp_c[1::2],
              b_proj.reshape(1, D_MODEL))
    Cf = _tc_project(wg_c, pool_c.reshape(N_C // CHUNK, PACKED, CHUNK),
                     *w_args, N_C // TN, 0)
    Qf = _tc_project(wg_q,
                     pool_q.reshape((N_TOK - N_C) // CHUNK, PACKED, CHUNK),
                     *w_args, (N_TOK - N_C) // TN, 0)
    return (Cf.reshape(B, LC, D_MODEL), Qf.reshape(B, LQ, D_MODEL))


# speedup vs baseline: 1.8985x; 1.0014x over previous
import functools

import jax
import jax.numpy as jnp
from jax import lax
from jax.experimental import pallas as pl
from jax.experimental.pallas import tpu as pltpu
from jax.experimental.pallas import tpu_sc as plsc

B = 1024
LC = 200
LQ = 50
LW = 16
WORD_DIM = 128
CHAR_DIM = 64
CHAR_VOCAB = 1000
D_MODEL = 128

N_TOK = B * (LC + LQ)
N_C = B * LC
NW = 32
PER_W = N_TOK // NW
CHUNK = 160
N_IT = PER_W // CHUNK
NCH_C = N_C // CHUNK
GHALF = CHUNK // 2
PACKED = CHAR_DIM // 2
ROWSTRIDE = PACKED + 1
CIDSTRIDE = LW + 1
PCH = PACKED * CHUNK


def _pack_pairs(tab):
    t16 = lax.bitcast_convert_type(tab.astype(jnp.bfloat16), jnp.uint16)
    return lax.bitcast_convert_type(
        t16[:, 0::2].astype(jnp.uint32)
        | (t16[:, 1::2].astype(jnp.uint32) << 16), jnp.int32)


def _sc_gather_pool(wid3, cid2, word_table, ctab_p, n_tok, off_ch):
    info = plsc.get_sparse_core_info()
    nc = info.num_cores
    n_it = n_tok // (NW * CHUNK)

    @functools.partial(
        pl.kernel,
        mesh=plsc.VectorSubcoreMesh(core_axis_name="c", subcore_axis_name="s"),
        compiler_params=pltpu.CompilerParams(needs_layout_passes=False),
        out_type=[
            jax.ShapeDtypeStruct((n_tok, WORD_DIM), jnp.float32),
            jax.ShapeDtypeStruct((n_tok * PACKED,), jnp.int32),
        ],
        scratch_types=[
            pltpu.VMEM((CHAR_VOCAB * ROWSTRIDE,), jnp.int32),
            pltpu.VMEM((2, 2, GHALF), jnp.int32),
            pltpu.VMEM((2, CHUNK, LW), jnp.int32),
            pltpu.VMEM((2, CHUNK, WORD_DIM), jnp.float32),
            pltpu.VMEM((2, PCH), jnp.int32),
            pltpu.SemaphoreType.DMA,
            pltpu.SemaphoreType.DMA,
            pltpu.SemaphoreType.DMA,
            pltpu.SemaphoreType.DMA,
            pltpu.SemaphoreType.DMA,
            pltpu.SemaphoreType.DMA,
            pltpu.SemaphoreType.DMA,
            pltpu.SemaphoreType.DMA,
            pltpu.SemaphoreType.DMA,
            pltpu.SemaphoreType.DMA,
        ],
    )
    def k(wid_hbm, cid_hbm, wtab_hbm, ctab_hbm,
          wg_hbm, pool_hbm,
          ctab_v, wid_v, cid_v, rows_v, pool_v,
          s_wid0, s_wid1, s_cid0, s_cid1, s_rows0, s_rows1,
          s_wout0, s_wout1, s_pout0, s_pout1):
        s_wid = (s_wid0, s_wid1)
        s_cid = (s_cid0, s_cid1)
        s_rows = (s_rows0, s_rows1)
        s_wout = (s_wout0, s_wout1)
        s_pout = (s_pout0, s_pout1)
        wid = lax.axis_index("s") * nc + lax.axis_index("c")
        ci0 = wid * n_it
        pltpu.sync_copy(ctab_hbm, ctab_v)
        iota16 = lax.iota(jnp.int32, 16)

        def start_in(ci, b):
            pltpu.async_copy(wid_hbm.at[off_ch + ci], wid_v.at[b], s_wid[b])
            pltpu.async_copy(
                cid_hbm.at[pl.ds((off_ch + ci) * CHUNK, CHUNK)],
                cid_v.at[b], s_cid[b])

        def wait_in(b):
            pltpu.make_async_copy(wid_hbm.at[0], wid_v.at[b], s_wid[b]).wait()
            pltpu.make_async_copy(cid_hbm.at[pl.ds(0, CHUNK)],
                                  cid_v.at[b], s_cid[b]).wait()

        def wait_out(b):
            pltpu.make_async_copy(rows_v.at[b],
                                  wg_hbm.at[pl.ds(0, CHUNK)], s_wout[b]).wait()
            pltpu.make_async_copy(pool_v.at[b],
                                  pool_hbm.at[pl.ds(0, PCH)], s_pout[b]).wait()

        start_in(ci0, 0)

        @pl.loop(0, n_it, step=2)
        def _outer(g):
            for b in (0, 1):
                it = g + b
                ci = ci0 + it
                base = ci * CHUNK
                wait_in(b)

                @pl.when(it + 1 < n_it)
                def _pf():
                    start_in(ci + 1, 1 - b)

                @pl.when(it >= 2)
                def _drain():
                    wait_out(b)

                cps = [pltpu.async_copy(
                    wtab_hbm.at[wid_v.at[b, h]],
                    rows_v.at[b, pl.ds(h * GHALF, GHALF)], s_rows[b])
                    for h in (0, 1)]

                def tb_body(tb, c2):
                    t0 = tb * 16
                    rowi = iota16 + t0
                    cids = [plsc.load_gather(
                        cid_v.at[b], [rowi, (iota16 + j) & (LW - 1)])
                        * ROWSTRIDE for j in range(LW)]
                    for p in range(PACKED):
                        m = plsc.bitcast(
                            plsc.load_gather(ctab_v, [cids[0] + p]),
                            jnp.bfloat16)
                        for j in range(1, LW):
                            m = jnp.maximum(m, plsc.bitcast(
                                plsc.load_gather(ctab_v, [cids[j] + p]),
                                jnp.bfloat16))
                        pool_v[b, pl.ds(p * CHUNK + t0, 16)] = (
                            plsc.bitcast(m, jnp.int32))
                    return c2

                lax.fori_loop(0, CHUNK // 16, tb_body, 0)
                for cp in cps:
                    cp.wait()
                pltpu.async_copy(rows_v.at[b], wg_hbm.at[pl.ds(base, CHUNK)],
                                 s_wout[b])
                pltpu.async_copy(pool_v.at[b],
                                 pool_hbm.at[pl.ds(ci * PCH, PCH)], s_pout[b])

        for b in (0, 1):
            wait_out(b)

    return k(wid3, cid2, word_table, ctab_p)


def _lo_f32(w):
    return lax.bitcast_convert_type(w << 16, jnp.float32)


def _hi_f32(w):
    return lax.bitcast_convert_type(w & jnp.int32(-65536), jnp.float32)


def _tc_project(wgath, pool3d, Ww, Wce, Wco, b2, nblk, off):
    CPB = 16
    TN = CPB * CHUNK

    def mm(wg_ref, pt_ref, ww_ref, wce_ref, wco_ref, b_ref, out_ref):
        word = jnp.dot(wg_ref[...], ww_ref[...],
                       preferred_element_type=jnp.float32) + b_ref[...]
        dn = (((0,), (0,)), ((), ()))
        for c in range(CPB):
            pt = pt_ref[c]
            ch = lax.dot_general(_lo_f32(pt), wce_ref[...],
                                 dimension_numbers=dn,
                                 preferred_element_type=jnp.float32)
            ch += lax.dot_general(_hi_f32(pt), wco_ref[...],
                                  dimension_numbers=dn,
                                  preferred_element_type=jnp.float32)
            out_ref[pl.ds(c * CHUNK, CHUNK), :] = (
                word[c * CHUNK:(c + 1) * CHUNK, :] + ch)

    return pl.pallas_call(
        mm,
        grid=(nblk,),
        in_specs=[
            pl.BlockSpec((TN, WORD_DIM), lambda i: (i + off, 0)),
            pl.BlockSpec((CPB, PACKED, CHUNK), lambda i: (i + off, 0, 0)),
            pl.BlockSpec((WORD_DIM, D_MODEL), lambda i: (0, 0)),
            pl.BlockSpec((PACKED, D_MODEL), lambda i: (0, 0)),
            pl.BlockSpec((PACKED, D_MODEL), lambda i: (0, 0)),
            pl.BlockSpec((1, D_MODEL), lambda i: (0, 0)),
        ],
        out_specs=pl.BlockSpec((TN, D_MODEL), lambda i: (i, 0)),
        out_shape=jax.ShapeDtypeStruct((nblk * TN, D_MODEL), jnp.float32),
    )(wgath, pool3d, Ww, Wce, Wco, b2)


def kernel(Cwid, Ccid, Qwid, Qcid, word_table, char_table, W_proj, b_proj):
    cwid3 = Cwid.reshape(NCH_C, 2, GHALF).astype(jnp.int32)
    qwid3 = Qwid.reshape(N_TOK // CHUNK - NCH_C, 2, GHALF).astype(jnp.int32)
    ccid2 = Ccid.reshape(N_C, LW).astype(jnp.int32)
    qcid2 = Qcid.reshape(N_TOK - N_C, LW).astype(jnp.int32)
    ctp = jnp.pad(_pack_pairs(char_table),
                  ((0, 0), (0, ROWSTRIDE - PACKED))).reshape(-1)

    wg_c, pool_c = _sc_gather_pool(cwid3, ccid2, word_table, ctp, N_C, 0)
    wg_q, pool_q = _sc_gather_pool(qwid3, qcid2, word_table, ctp,
                                   N_TOK - N_C, 0)

    Wp_c = W_proj[WORD_DIM:]
    TN = 16 * CHUNK
    w_args = (W_proj[:WORD_DIM], Wp_c[0::2], Wp_c[1::2],
              b_proj.reshape(1, D_MODEL))
    Cf = _tc_project(wg_c, pool_c.reshape(N_C // CHUNK, PACKED, CHUNK),
                     *w_args, N_C // TN, 0)
    Qf = _tc_project(wg_q,
                     pool_q.reshape((N_TOK - N_C) // CHUNK, PACKED, CHUNK),
                     *w_args, (N_TOK - N_C) // TN, 0)
    return (Cf.reshape(B, LC, D_MODEL), Qf.reshape(B, LQ, D_MODEL))
